# baseline (speedup 1.0000x reference)
import functools

import jax
import jax.numpy as jnp
from jax import lax
from jax.experimental import pallas as pl
from jax.experimental.pallas import tpu as pltpu
from jax.experimental.pallas import tpu_sc as plsc

N_FIELDS = 26
VOCAB = 100000
EMBED = 32
BATCH = 16384

NC, NS, L = 2, 16, 16
NW = NC * NS
TOTAL = BATCH * N_FIELDS
PER_W = TOTAL // NW
CHUNK = 1024
N_CH = PER_W // CHUNK
NBUF = 3

PLANES = N_FIELDS * EMBED
VFULL = VOCAB // 128
VREM = VOCAB - VFULL * 128
G_ROWS = N_FIELDS * VOCAB * EMBED // 128
FULL_BLOCKS = N_FIELDS * VFULL

_mesh = plsc.VectorSubcoreMesh(
    core_axis_name="c", subcore_axis_name="s", num_cores=NC, num_subcores=NS
)


@functools.partial(
    pl.kernel,
    out_type=jax.ShapeDtypeStruct((G_ROWS, 128), jnp.float32),
    mesh=_mesh,
    compiler_params=pltpu.CompilerParams(needs_layout_passes=False),
    scratch_types=[
        pltpu.VMEM((EMBED, 128), jnp.float32),
        pltpu.VMEM((EMBED, 128), jnp.float32),
        pltpu.SemaphoreType.DMA,
        pltpu.SemaphoreType.DMA,
    ],
)
def _sc_detile(tt_hbm, rem_hbm, g_hbm, in_v, out_v, i_sem, o_sem):
    wid = lax.axis_index("s") * NC + lax.axis_index("c")

    def in_slice(f, c):
        return tt_hbm.at[pl.ds(f * EMBED, EMBED), pl.ds(c * 128, 128)]

    def emit_vector(buf, e, v16, obuf):
        iota = lax.iota(jnp.int32, L)
        x = buf[e, pl.ds(v16 * 16, L)]
        plsc.store_scatter(obuf, [iota // 4 + v16 * 4, (iota % 4) * 32 + e], x)

    def block_body(i, carry):
        beta = i * NW + wid

        @pl.when(beta < FULL_BLOCKS)
        def _():
            f = beta // VFULL
            c = beta % VFULL
            pltpu.async_copy(in_slice(f, c), in_v, i_sem).wait()
            for e in range(EMBED):
                for v16 in range(8):
                    emit_vector(in_v, e, v16, out_v)
            g0 = f * (VOCAB * EMBED // 128) + c * EMBED
            pltpu.async_copy(
                out_v, g_hbm.at[pl.ds(g0, EMBED)], o_sem
            ).wait()

        return carry

    n_iter = (FULL_BLOCKS + NW - 1) // NW
    lax.fori_loop(0, n_iter, block_body, 0)

    @pl.when(wid < N_FIELDS)
    def _():
        f = wid
        nrem = VREM * EMBED // 128
        pltpu.async_copy(
            rem_hbm.at[pl.ds(f * nrem, nrem)], in_v.at[pl.ds(0, nrem)], i_sem
        ).wait()
        g0 = f * (VOCAB * EMBED // 128) + VFULL * EMBED
        pltpu.async_copy(
            in_v.at[pl.ds(0, nrem)], g_hbm.at[pl.ds(g0, nrem)], o_sem
        ).wait()


@functools.partial(
    pl.kernel,
    out_type=jax.ShapeDtypeStruct((TOTAL, EMBED), jnp.float32),
    mesh=_mesh,
    compiler_params=pltpu.CompilerParams(use_tc_tiling_on_sc=False),
    scratch_types=[
        pltpu.VMEM((PER_W,), jnp.int32),
        pltpu.VMEM((NBUF, CHUNK, EMBED), jnp.float32),
        [pltpu.SemaphoreType.DMA] * NBUF,
        [pltpu.SemaphoreType.DMA] * NBUF,
    ],
)
def _sc_gather(tables_hbm, tokens_hbm, out_hbm, idx_v, rows_v, g_sems, o_sems):
    wid = lax.axis_index("s") * NC + lax.axis_index("c")
    base = wid * PER_W

    pltpu.sync_copy(tokens_hbm.at[pl.ds(base, PER_W)], idx_v)

    def body(j, carry):
        p0 = j * L
        lane = p0 + lax.iota(jnp.int32, L)
        field = lax.rem(lane, N_FIELDS)
        idx_v[pl.ds(p0, L)] = idx_v[pl.ds(p0, L)] + field * VOCAB
        return carry

    lax.fori_loop(0, PER_W // L, body, 0)

    def gather(c):
        return pltpu.async_copy(
            tables_hbm.at[idx_v.at[pl.ds(c * CHUNK, CHUNK)]],
            rows_v.at[c % NBUF],
            g_sems[c % NBUF],
        )

    def write_out(c):
        return pltpu.async_copy(
            rows_v.at[c % NBUF],
            out_hbm.at[pl.ds(base + c * CHUNK, CHUNK)],
            o_sems[c % NBUF],
        )

    g_h = [None] * N_CH
    o_h = [None] * N_CH
    for c in range(min(NBUF - 1, N_CH)):
        g_h[c] = gather(c)
    for c in range(N_CH):
        if c >= 1:
            o_h[c - 1].wait()
        nxt = c + NBUF - 1
        if nxt < N_CH:
            g_h[nxt] = gather(nxt)
        g_h[c].wait()
        o_h[c] = write_out(c)
    o_h[N_CH - 1].wait()


def kernel(tokens, tables):
    f = tables.shape[0]
    d = tables.shape[-1]
    tt = tables.transpose(0, 2, 1).reshape(f * d, tables.shape[1])
    rem = tables[:, VFULL * 128 :, :].reshape(N_FIELDS * VREM * EMBED // 128, 128)
    g = _sc_detile(tt, rem)
    g2 = g.reshape(f * tables.shape[1], d)
    tok_flat = tokens.reshape(-1).astype(jnp.int32)
    out = _sc_gather(g2, tok_flat)
    return out.reshape(tokens.shape[0], f, d)


# speedup vs baseline: 1.3626x; 1.3626x over previous
import functools

import jax
import jax.numpy as jnp
from jax import lax
from jax.experimental import pallas as pl
from jax.experimental.pallas import tpu as pltpu
from jax.experimental.pallas import tpu_sc as plsc

N_FIELDS = 26
VOCAB = 100000
EMBED = 32
BATCH = 16384

NC, NS, L = 2, 16, 16
NW = NC * NS
TOTAL = BATCH * N_FIELDS
PER_W = TOTAL // NW
CHUNK = 1024
N_CH = PER_W // CHUNK
NBUF = 3

PLANES = N_FIELDS * EMBED
VFULL = VOCAB // 128
VREM = VOCAB - VFULL * 128
G_ROWS = N_FIELDS * VOCAB * EMBED // 128
FULL_BLOCKS = N_FIELDS * VFULL

_mesh = plsc.VectorSubcoreMesh(
    core_axis_name="c", subcore_axis_name="s", num_cores=NC, num_subcores=NS
)


@functools.partial(
    pl.kernel,
    out_type=jax.ShapeDtypeStruct((G_ROWS, 128), jnp.float32),
    mesh=_mesh,
    compiler_params=pltpu.CompilerParams(needs_layout_passes=False),
    scratch_types=[
        pltpu.VMEM((EMBED, 128), jnp.float32),
        pltpu.VMEM((EMBED, 128), jnp.float32),
        pltpu.VMEM((EMBED, 128), jnp.float32),
        pltpu.VMEM((EMBED, 128), jnp.float32),
        [pltpu.SemaphoreType.DMA] * 2,
        [pltpu.SemaphoreType.DMA] * 2,
    ],
)
def _sc_detile(tt_hbm, rem_hbm, g_hbm, in_a, in_b, out_a, out_b, i_sems, o_sems):
    wid = lax.axis_index("s") * NC + lax.axis_index("c")
    ins = [in_a, in_b]
    outs = [out_a, out_b]

    NFB = FULL_BLOCKS // NW
    TAIL = FULL_BLOCKS - NFB * NW

    def fc(i):
        beta = i * NW + wid
        return beta // VFULL, beta % VFULL

    def start_in(i, k):
        f, c = fc(i)
        return pltpu.async_copy(
            tt_hbm.at[pl.ds(f * EMBED, EMBED), pl.ds(c * 128, 128)],
            ins[k],
            i_sems[k],
        )

    def start_out(i, k):
        f, c = fc(i)
        g0 = f * (VOCAB * EMBED // 128) + c * EMBED
        return pltpu.async_copy(outs[k], g_hbm.at[pl.ds(g0, EMBED)], o_sems[k])

    def drain_in(k):
        pltpu.make_async_copy(
            tt_hbm.at[pl.ds(0, EMBED), pl.ds(0, 128)], ins[k], i_sems[k]
        ).wait()

    def drain_out(k):
        pltpu.make_async_copy(
            outs[k], g_hbm.at[pl.ds(0, EMBED)], o_sems[k]
        ).wait()

    iota = lax.iota(jnp.int32, L)
    ridx = [iota // 4 + v16 * 4 for v16 in range(8)]
    cbase = (iota % 4) * 32

    def compute(k):
        for e in range(EMBED):
            cidx = cbase + e
            for v16 in range(8):
                x = ins[k][e, pl.ds(v16 * 16, L)]
                plsc.store_scatter(outs[k], [ridx[v16], cidx], x)

    start_in(0, 0)
    start_in(1, 1)
    for k in (0, 1):
        drain_in(k)
        compute(k)
        start_in(2 + k, k)
        start_out(k, k)

    def pair(j, carry):
        for k in (0, 1):
            b = 2 * j + k
            drain_in(k)
            drain_out(k)
            compute(k)

            @pl.when(b + 2 < NFB)
            def _():
                start_in(b + 2, k)

            start_out(b, k)
        return carry

    lax.fori_loop(1, NFB // 2, pair, 0)
    drain_out(0)
    drain_out(1)

    @pl.when(wid < TAIL)
    def _():
        start_in(NFB, 0).wait()
        compute(0)
        start_out(NFB, 0).wait()

    @pl.when(wid < N_FIELDS)
    def _():
        f = wid
        nrem = VREM * EMBED // 128
        pltpu.async_copy(
            rem_hbm.at[pl.ds(f * nrem, nrem)], in_a.at[pl.ds(0, nrem)], i_sems[0]
        ).wait()
        g0 = f * (VOCAB * EMBED // 128) + VFULL * EMBED
        pltpu.async_copy(
            in_a.at[pl.ds(0, nrem)], g_hbm.at[pl.ds(g0, nrem)], o_sems[0]
        ).wait()


@functools.partial(
    pl.kernel,
    out_type=jax.ShapeDtypeStruct((TOTAL, EMBED), jnp.float32),
    mesh=_mesh,
    compiler_params=pltpu.CompilerParams(use_tc_tiling_on_sc=False),
    scratch_types=[
        pltpu.VMEM((PER_W,), jnp.int32),
        pltpu.VMEM((NBUF, CHUNK, EMBED), jnp.float32),
        [pltpu.SemaphoreType.DMA] * NBUF,
        [pltpu.SemaphoreType.DMA] * NBUF,
    ],
)
def _sc_gather(tables_hbm, tokens_hbm, out_hbm, idx_v, rows_v, g_sems, o_sems):
    wid = lax.axis_index("s") * NC + lax.axis_index("c")
    base = wid * PER_W

    pltpu.sync_copy(tokens_hbm.at[pl.ds(base, PER_W)], idx_v)

    def body(j, carry):
        p0 = j * L
        lane = p0 + lax.iota(jnp.int32, L)
        field = lax.rem(lane, N_FIELDS)
        idx_v[pl.ds(p0, L)] = idx_v[pl.ds(p0, L)] + field * VOCAB
        return carry

    lax.fori_loop(0, PER_W // L, body, 0)

    def gather(c):
        return pltpu.async_copy(
            tables_hbm.at[idx_v.at[pl.ds(c * CHUNK, CHUNK)]],
            rows_v.at[c % NBUF],
            g_sems[c % NBUF],
        )

    def write_out(c):
        return pltpu.async_copy(
            rows_v.at[c % NBUF],
            out_hbm.at[pl.ds(base + c * CHUNK, CHUNK)],
            o_sems[c % NBUF],
        )

    g_h = [None] * N_CH
    o_h = [None] * N_CH
    for c in range(min(NBUF - 1, N_CH)):
        g_h[c] = gather(c)
    for c in range(N_CH):
        if c >= 1:
            o_h[c - 1].wait()
        nxt = c + NBUF - 1
        if nxt < N_CH:
            g_h[nxt] = gather(nxt)
        g_h[c].wait()
        o_h[c] = write_out(c)
    o_h[N_CH - 1].wait()


def kernel(tokens, tables):
    f = tables.shape[0]
    d = tables.shape[-1]
    tt = tables.transpose(0, 2, 1).reshape(f * d, tables.shape[1])
    rem = tables[:, VFULL * 128 :, :].reshape(N_FIELDS * VREM * EMBED // 128, 128)
    g = _sc_detile(tt, rem)
    g2 = g.reshape(f * tables.shape[1], d)
    tok_flat = tokens.reshape(-1).astype(jnp.int32)
    out = _sc_gather(g2, tok_flat)
    return out.reshape(tokens.shape[0], f, d)


# speedup vs baseline: 1.3654x; 1.0020x over previous
import functools

import jax
import jax.numpy as jnp
from jax import lax
from jax.experimental import pallas as pl
from jax.experimental.pallas import tpu as pltpu
from jax.experimental.pallas import tpu_sc as plsc

N_FIELDS = 26
VOCAB = 100000
EMBED = 32
BATCH = 16384

NC, NS, L = 2, 16, 16
NW = NC * NS
TOTAL = BATCH * N_FIELDS
PER_W = TOTAL // NW
CHUNK = 1024
N_CH = PER_W // CHUNK
NBUF = 3

PLANES = N_FIELDS * EMBED
VFULL = VOCAB // 128
VREM = VOCAB - VFULL * 128
G_ROWS = N_FIELDS * VOCAB * EMBED // 128
FULL_BLOCKS = N_FIELDS * VFULL

_mesh = plsc.VectorSubcoreMesh(
    core_axis_name="c", subcore_axis_name="s", num_cores=NC, num_subcores=NS
)


@functools.partial(
    pl.kernel,
    out_type=jax.ShapeDtypeStruct((G_ROWS, 128), jnp.float32),
    mesh=_mesh,
    compiler_params=pltpu.CompilerParams(needs_layout_passes=False),
    scratch_types=[
        pltpu.VMEM((EMBED, 128), jnp.float32),
        pltpu.VMEM((EMBED, 128), jnp.float32),
        pltpu.VMEM((EMBED, 128), jnp.float32),
        pltpu.VMEM((EMBED, 128), jnp.float32),
        [pltpu.SemaphoreType.DMA] * 2,
        [pltpu.SemaphoreType.DMA] * 2,
    ],
)
def _sc_detile(tt_hbm, rem_hbm, g_hbm, in_a, in_b, out_a, out_b, i_sems, o_sems):
    wid = lax.axis_index("s") * NC + lax.axis_index("c")
    ins = [in_a, in_b]
    outs = [out_a, out_b]

    NFB = FULL_BLOCKS // NW
    TAIL = FULL_BLOCKS - NFB * NW

    def fc(i):
        beta = i * NW + wid
        return beta // VFULL, beta % VFULL

    def start_in(i, k):
        f, c = fc(i)
        return pltpu.async_copy(
            tt_hbm.at[pl.ds(f * EMBED, EMBED), pl.ds(c * 128, 128)],
            ins[k],
            i_sems[k],
        )

    def start_out(i, k):
        f, c = fc(i)
        g0 = f * (VOCAB * EMBED // 128) + c * EMBED
        return pltpu.async_copy(outs[k], g_hbm.at[pl.ds(g0, EMBED)], o_sems[k])

    def drain_in(k):
        pltpu.make_async_copy(
            tt_hbm.at[pl.ds(0, EMBED), pl.ds(0, 128)], ins[k], i_sems[k]
        ).wait()

    def drain_out(k):
        pltpu.make_async_copy(
            outs[k], g_hbm.at[pl.ds(0, EMBED)], o_sems[k]
        ).wait()

    iota = lax.iota(jnp.int32, L)
    cbase = (iota % 4) * 32

    def compute(k):
        def vbody(v16, carry):
            r = iota // 4 + v16 * 4
            for e in range(EMBED):
                x = ins[k][e, pl.ds(v16 * 16, L)]
                plsc.store_scatter(outs[k], [r, cbase + e], x)
            return carry

        lax.fori_loop(0, 8, vbody, 0)

    start_in(0, 0)
    start_in(1, 1)
    for k in (0, 1):
        drain_in(k)
        compute(k)
        start_in(2 + k, k)
        start_out(k, k)

    def pair(j, carry):
        for k in (0, 1):
            b = 2 * j + k
            drain_in(k)
            drain_out(k)
            compute(k)

            @pl.when(b + 2 < NFB)
            def _():
                start_in(b + 2, k)

            start_out(b, k)
        return carry

    lax.fori_loop(1, NFB // 2, pair, 0)
    drain_out(0)
    drain_out(1)

    @pl.when(wid < TAIL)
    def _():
        start_in(NFB, 0).wait()
        compute(0)
        start_out(NFB, 0).wait()

    @pl.when(wid < N_FIELDS)
    def _():
        f = wid
        nrem = VREM * EMBED // 128
        pltpu.async_copy(
            rem_hbm.at[pl.ds(f * nrem, nrem)], in_a.at[pl.ds(0, nrem)], i_sems[0]
        ).wait()
        g0 = f * (VOCAB * EMBED // 128) + VFULL * EMBED
        pltpu.async_copy(
            in_a.at[pl.ds(0, nrem)], g_hbm.at[pl.ds(g0, nrem)], o_sems[0]
        ).wait()


@functools.partial(
    pl.kernel,
    out_type=jax.ShapeDtypeStruct((TOTAL, EMBED), jnp.float32),
    mesh=_mesh,
    compiler_params=pltpu.CompilerParams(use_tc_tiling_on_sc=False),
    scratch_types=[
        pltpu.VMEM((PER_W,), jnp.int32),
        pltpu.VMEM((NBUF, CHUNK, EMBED), jnp.float32),
        [pltpu.SemaphoreType.DMA] * NBUF,
        [pltpu.SemaphoreType.DMA] * NBUF,
    ],
)
def _sc_gather(tables_hbm, tokens_hbm, out_hbm, idx_v, rows_v, g_sems, o_sems):
    wid = lax.axis_index("s") * NC + lax.axis_index("c")
    base = wid * PER_W

    pltpu.sync_copy(tokens_hbm.at[pl.ds(base, PER_W)], idx_v)

    def body(j, carry):
        p0 = j * L
        lane = p0 + lax.iota(jnp.int32, L)
        field = lax.rem(lane, N_FIELDS)
        idx_v[pl.ds(p0, L)] = idx_v[pl.ds(p0, L)] + field * VOCAB
        return carry

    lax.fori_loop(0, PER_W // L, body, 0)

    def gather(c):
        return pltpu.async_copy(
            tables_hbm.at[idx_v.at[pl.ds(c * CHUNK, CHUNK)]],
            rows_v.at[c % NBUF],
            g_sems[c % NBUF],
        )

    def write_out(c):
        return pltpu.async_copy(
            rows_v.at[c % NBUF],
            out_hbm.at[pl.ds(base + c * CHUNK, CHUNK)],
            o_sems[c % NBUF],
        )

    g_h = [None] * N_CH
    o_h = [None] * N_CH
    for c in range(min(NBUF - 1, N_CH)):
        g_h[c] = gather(c)
    for c in range(N_CH):
        if c >= 1:
            o_h[c - 1].wait()
        nxt = c + NBUF - 1
        if nxt < N_CH:
            g_h[nxt] = gather(nxt)
        g_h[c].wait()
        o_h[c] = write_out(c)
    o_h[N_CH - 1].wait()


def kernel(tokens, tables):
    f = tables.shape[0]
    d = tables.shape[-1]
    tt = tables.transpose(0, 2, 1).reshape(f * d, tables.shape[1])
    rem = tables[:, VFULL * 128 :, :].reshape(N_FIELDS * VREM * EMBED // 128, 128)
    g = _sc_detile(tt, rem)
    g2 = g.reshape(f * tables.shape[1], d)
    tok_flat = tokens.reshape(-1).astype(jnp.int32)
    out = _sc_gather(g2, tok_flat)
    return out.reshape(tokens.shape[0], f, d)
